# baseline (speedup 1.0000x reference)
import jax
import jax.numpy as jnp
from jax.experimental import pallas as pl
from jax.experimental.pallas import tpu as pltpu
from jax.experimental.pallas import tpu_sc as plsc


_WINDOW = 128
_SPLIT = 2


def kernel(x, embed_weight):
    b, s = x.shape
    n = b * s
    v, d = embed_weight.shape
    dh = d // _SPLIT
    n2 = n * _SPLIT
    table2 = embed_weight.reshape(v * _SPLIT, dh)
    flat = x.reshape(n).astype(jnp.int32)
    idx2 = (flat[:, None] * _SPLIT + jnp.arange(_SPLIT, dtype=jnp.int32)
            ).reshape(1, n2)

    mesh = plsc.VectorSubcoreMesh(core_axis_name="c", subcore_axis_name="s")

    @jax.jit
    def gather(table, indices):
        @pl.kernel(
            out_type=jax.ShapeDtypeStruct((n2, dh), table.dtype),
            mesh=mesh,
        )
        def k(table_hbm, i_hbm, o_hbm):
            def body(i_vmem, o_vmem):
                pltpu.sync_copy(table_hbm.at[i_vmem.at[0]], o_vmem)

            pltpu.emit_pipeline(
                body,
                grid=(n2 // _WINDOW,),
                in_specs=[pl.BlockSpec((1, _WINDOW), lambda i: (0, i))],
                out_specs=[pl.BlockSpec((_WINDOW, dh), lambda i: (i, 0))],
                core_axis_name=("c", "s"),
                dimension_semantics=(pltpu.PARALLEL,),
            )(i_hbm, o_hbm)

        return k(table, indices)

    return gather(table2, idx2).reshape(b, s, d)


# speedup vs baseline: 3.9632x; 3.9632x over previous
import jax
import jax.numpy as jnp
from jax import lax
from jax.experimental import pallas as pl
from jax.experimental.pallas import tpu as pltpu
from jax.experimental.pallas import tpu_sc as plsc

_NUM_CORES = 2
_NUM_SUBCORES = 16
_CHUNK = 64


def kernel(x, embed_weight):
    b, s = x.shape
    n = b * s
    v, d = embed_weight.shape
    nw = _NUM_CORES * _NUM_SUBCORES
    bw = n // nw
    nchunk = bw // _CHUNK
    flat = x.reshape(n).astype(jnp.int32)

    mesh = plsc.VectorSubcoreMesh(core_axis_name="c", subcore_axis_name="s")

    @pl.kernel(
        out_type=jax.ShapeDtypeStruct((b, s, d), embed_weight.dtype),
        mesh=mesh,
        scratch_types=[
            pltpu.VMEM((bw,), jnp.int32),
            pltpu.VMEM((_CHUNK, d), jnp.float32),
            pltpu.VMEM((_CHUNK, d), jnp.float32),
            pltpu.SemaphoreType.DMA,
            pltpu.SemaphoreType.DMA,
        ],
    )
    def k(table_hbm, i_hbm, o_hbm, idx_v, rows0, rows1, gsem, wsem):
        wid = lax.axis_index("s") * _NUM_CORES + lax.axis_index("c")
        base = wid * bw
        bi = base // s
        col0 = base % s
        pltpu.sync_copy(i_hbm.at[pl.ds(base, bw)], idx_v)

        bufs = (rows0, rows1)
        gathers = [None] * nchunk
        writes = [None] * nchunk
        gathers[0] = pltpu.async_copy(
            table_hbm.at[idx_v.at[pl.ds(0, _CHUNK)]], bufs[0], gsem)
        for c in range(nchunk):
            buf = bufs[c % 2]
            gathers[c].wait()
            if c >= 1:
                writes[c - 1].wait()
            if c + 1 < nchunk:
                gathers[c + 1] = pltpu.async_copy(
                    table_hbm.at[idx_v.at[pl.ds((c + 1) * _CHUNK, _CHUNK)]],
                    bufs[(c + 1) % 2], gsem)
            writes[c] = pltpu.async_copy(
                buf, o_hbm.at[bi, pl.ds(col0 + c * _CHUNK, _CHUNK)], wsem)
        writes[nchunk - 1].wait()

    return k(embed_weight, flat)


# speedup vs baseline: 4.0479x; 1.0214x over previous
import jax
import jax.numpy as jnp
from jax import lax
from jax.experimental import pallas as pl
from jax.experimental.pallas import tpu as pltpu
from jax.experimental.pallas import tpu_sc as plsc

_NUM_CORES = 2
_NUM_SUBCORES = 16
_CHUNK = 32
_NBUF = 4


def kernel(x, embed_weight):
    b, s = x.shape
    n = b * s
    v, d = embed_weight.shape
    nw = _NUM_CORES * _NUM_SUBCORES
    bw = n // nw
    nchunk = bw // _CHUNK

    mesh = plsc.VectorSubcoreMesh(core_axis_name="c", subcore_axis_name="s")

    @pl.kernel(
        out_type=jax.ShapeDtypeStruct((b, s, d), embed_weight.dtype),
        mesh=mesh,
        scratch_types=[
            pltpu.VMEM((bw,), jnp.int32),
        ] + [pltpu.VMEM((_CHUNK, d), jnp.float32) for _ in range(_NBUF)] + [
            pltpu.SemaphoreType.DMA,
            pltpu.SemaphoreType.DMA,
        ],
    )
    def k(table_hbm, i_hbm, o_hbm, idx_v, *rest):
        bufs = rest[:_NBUF]
        gsem, wsem = rest[_NBUF:]
        wid = lax.axis_index("s") * _NUM_CORES + lax.axis_index("c")
        base = wid * bw
        bi = base // s
        col0 = base % s
        pltpu.sync_copy(i_hbm.at[bi, pl.ds(col0, bw)], idx_v)

        def start_gather(c):
            return pltpu.async_copy(
                table_hbm.at[idx_v.at[pl.ds(c * _CHUNK, _CHUNK)]],
                bufs[c % _NBUF], gsem)

        gathers = [None] * nchunk
        writes = [None] * nchunk
        for c in range(_NBUF - 1):
            gathers[c] = start_gather(c)
        for c in range(nchunk):
            gathers[c].wait()
            writes[c] = pltpu.async_copy(
                bufs[c % _NBUF],
                o_hbm.at[bi, pl.ds(col0 + c * _CHUNK, _CHUNK)], wsem)
            if c + _NBUF - 1 < nchunk:
                if c >= 1:
                    writes[c - 1].wait()
                gathers[c + _NBUF - 1] = start_gather(c + _NBUF - 1)
        for c in range(max(0, nchunk - _NBUF), nchunk):
            writes[c].wait()

    return k(embed_weight, x.astype(jnp.int32))


# speedup vs baseline: 4.0621x; 1.0035x over previous
import jax
import jax.numpy as jnp
from jax import lax
from jax.experimental import pallas as pl
from jax.experimental.pallas import tpu as pltpu
from jax.experimental.pallas import tpu_sc as plsc

_NUM_CORES = 2
_NUM_SUBCORES = 16
_CHUNK = 32
_NBUF = 5


def kernel(x, embed_weight):
    b, s = x.shape
    n = b * s
    v, d = embed_weight.shape
    nw = _NUM_CORES * _NUM_SUBCORES
    bw = n // nw
    nchunk = bw // _CHUNK

    mesh = plsc.VectorSubcoreMesh(core_axis_name="c", subcore_axis_name="s")

    @pl.kernel(
        out_type=jax.ShapeDtypeStruct((b, s, d), embed_weight.dtype),
        mesh=mesh,
        scratch_types=[
            pltpu.VMEM((bw,), jnp.int32),
        ] + [pltpu.VMEM((_CHUNK, d), jnp.float32) for _ in range(_NBUF)] + [
            pltpu.SemaphoreType.DMA,
            pltpu.SemaphoreType.DMA,
        ],
    )
    def k(table_hbm, i_hbm, o_hbm, idx_v, *rest):
        bufs = rest[:_NBUF]
        gsem, wsem = rest[_NBUF:]
        wid = lax.axis_index("s") * _NUM_CORES + lax.axis_index("c")
        base = wid * bw
        bi = base // s
        col0 = base % s
        pltpu.sync_copy(i_hbm.at[bi, pl.ds(col0, bw)], idx_v)

        def start_gather(c):
            return pltpu.async_copy(
                table_hbm.at[idx_v.at[pl.ds(c * _CHUNK, _CHUNK)]],
                bufs[c % _NBUF], gsem)

        gathers = [None] * nchunk
        writes = [None] * nchunk
        for c in range(_NBUF - 1):
            gathers[c] = start_gather(c)
        for c in range(nchunk):
            gathers[c].wait()
            writes[c] = pltpu.async_copy(
                bufs[c % _NBUF],
                o_hbm.at[bi, pl.ds(col0 + c * _CHUNK, _CHUNK)], wsem)
            if c + _NBUF - 1 < nchunk:
                if c >= 1:
                    writes[c - 1].wait()
                gathers[c + _NBUF - 1] = start_gather(c + _NBUF - 1)
        for c in range(max(0, nchunk - _NBUF), nchunk):
            writes[c].wait()

    return k(embed_weight, x.astype(jnp.int32))
